# baseline (speedup 1.0000x reference)
import functools

import jax
import jax.numpy as jnp
from jax import lax
from jax.experimental import pallas as pl
from jax.experimental.pallas import tpu as pltpu
from jax.experimental.pallas import tpu_sc as plsc

E_ROWS = 150000
IN_DIM = 512
OUT_DIM = 64
PAD_DIM = 128
B = 100000


_MM_ROWS = 3000


_MM_HALF = E_ROWS // 2
_MM_STEPS = _MM_HALF // _MM_ROWS if _MM_HALF % _MM_ROWS == 0 else None


def _mm_body(xa_ref, xb_ref, wt_ref, b_ref, o_ref):
    ra = (
        jnp.dot(xa_ref[...], wt_ref[...], preferred_element_type=jnp.float32)
        + b_ref[...]
    )
    rb = (
        jnp.dot(xb_ref[...], wt_ref[...], preferred_element_type=jnp.float32)
        + b_ref[...]
    )
    o_ref[...] = jnp.concatenate([ra, rb], axis=1)


def _transform_table(x, wt, b2d):
    steps = _MM_HALF // _MM_ROWS
    return pl.pallas_call(
        _mm_body,
        grid=(steps,),
        in_specs=[
            pl.BlockSpec((_MM_ROWS, IN_DIM), lambda i: (i, 0)),
            pl.BlockSpec((_MM_ROWS, IN_DIM), lambda i, s=steps: (i + s, 0)),
            pl.BlockSpec((IN_DIM, OUT_DIM), lambda i: (0, 0)),
            pl.BlockSpec((1, OUT_DIM), lambda i: (0, 0)),
        ],
        out_specs=pl.BlockSpec((_MM_ROWS, PAD_DIM), lambda i: (i, 0)),
        out_shape=jax.ShapeDtypeStruct((_MM_HALF, PAD_DIM), jnp.float32),
    )(x, x, wt, b2d)


_CHUNK = 320
_DEPTH = 3
_W_FULL = 3136
_W_LAST = B - 31 * _W_FULL
_NCH = 10


def _load_q(idx_hbm, ibuf, off):
    pltpu.sync_copy(idx_hbm.at[pl.ds(off, _CHUNK)], ibuf)


def _gather_body(table_hbm, idx_hbm, out_hbm, i0, i1, i2, r0, r1, r2, s0, s1, s2):
    wid = lax.axis_index("s") * 2 + lax.axis_index("c")
    base = wid * _W_FULL
    last = jnp.where(wid == 31, _W_LAST, _W_FULL) - _CHUNK

    def off(k):
        return base + jnp.minimum(k * _CHUNK, last)

    ibufs, rbufs, sems = [i0, i1, i2], [r0, r1, r2], [s0, s1, s2]
    hs = [None] * _NCH
    for j in range(_DEPTH - 1):
        _load_q(idx_hbm, ibufs[j], off(j))
        hs[j] = pltpu.async_copy(table_hbm.at[ibufs[j]], rbufs[j], sems[j])
    for k in range(_NCH):
        kk = k + _DEPTH - 1
        if kk < _NCH:
            s = kk % _DEPTH
            _load_q(idx_hbm, ibufs[s], off(kk))
            hs[kk] = pltpu.async_copy(table_hbm.at[ibufs[s]], rbufs[s], sems[s])
        hs[k].wait()
        pltpu.sync_copy(rbufs[k % _DEPTH], out_hbm.at[pl.ds(off(k), _CHUNK)])


def _gather_rows(table, idx):
    mesh = plsc.VectorSubcoreMesh(core_axis_name="c", subcore_axis_name="s")
    k = functools.partial(
        pl.kernel,
        mesh=mesh,
        out_type=jax.ShapeDtypeStruct((B, OUT_DIM), jnp.float32),
        compiler_params=pltpu.CompilerParams(use_tc_tiling_on_sc=False),
        scratch_types=[
            pltpu.VMEM((_CHUNK,), jnp.int32),
            pltpu.VMEM((_CHUNK,), jnp.int32),
            pltpu.VMEM((_CHUNK,), jnp.int32),
            pltpu.VMEM((_CHUNK, OUT_DIM), jnp.float32),
            pltpu.VMEM((_CHUNK, OUT_DIM), jnp.float32),
            pltpu.VMEM((_CHUNK, OUT_DIM), jnp.float32),
            pltpu.SemaphoreType.DMA,
            pltpu.SemaphoreType.DMA,
            pltpu.SemaphoreType.DMA,
        ],
    )(_gather_body)
    return k(table, idx)


def kernel(edge_features, neighbors_edge_idxs, W, b):
    table = _transform_table(edge_features, W.T, b.reshape(1, OUT_DIM))
    table = table.reshape(E_ROWS, OUT_DIM)
    idx = neighbors_edge_idxs.astype(jnp.int32)
    q = idx + idx - jnp.where(idx >= _MM_HALF, 2 * _MM_HALF - 1, 0)
    return _gather_rows(table, q)


# speedup vs baseline: 1.2352x; 1.2352x over previous
import functools

import jax
import jax.numpy as jnp
from jax import lax
from jax.experimental import pallas as pl
from jax.experimental.pallas import tpu as pltpu
from jax.experimental.pallas import tpu_sc as plsc

E_ROWS = 150000
IN_DIM = 512
OUT_DIM = 64
PAD_DIM = 128
B = 100000


_MM_ROWS = 3000


_MM_HALF = E_ROWS // 2
_MM_STEPS = _MM_HALF // _MM_ROWS if _MM_HALF % _MM_ROWS == 0 else None


def _mm_body(xa_ref, xb_ref, wt_ref, b_ref, o_ref):
    ra = (
        jnp.dot(xa_ref[...], wt_ref[...], preferred_element_type=jnp.float32)
        + b_ref[...]
    )
    rb = (
        jnp.dot(xb_ref[...], wt_ref[...], preferred_element_type=jnp.float32)
        + b_ref[...]
    )
    o_ref[...] = jnp.concatenate([ra, rb], axis=1)


def _transform_table(x, wt, b2d):
    steps = _MM_HALF // _MM_ROWS
    return pl.pallas_call(
        _mm_body,
        grid=(steps,),
        in_specs=[
            pl.BlockSpec((_MM_ROWS, IN_DIM), lambda i: (i, 0)),
            pl.BlockSpec((_MM_ROWS, IN_DIM), lambda i, s=steps: (i + s, 0)),
            pl.BlockSpec((IN_DIM, OUT_DIM), lambda i: (0, 0)),
            pl.BlockSpec((1, OUT_DIM), lambda i: (0, 0)),
        ],
        out_specs=pl.BlockSpec((_MM_ROWS, PAD_DIM), lambda i: (i, 0)),
        out_shape=jax.ShapeDtypeStruct((_MM_HALF, PAD_DIM), jnp.float32),
    )(x, x, wt, b2d)


_CHUNK = 320
_DEPTH = 3
_W_FULL = 3136
_W_LAST = B - 31 * _W_FULL
_NCH = 10


def _load_q(idx_hbm, ibuf, off):
    pltpu.sync_copy(idx_hbm.at[pl.ds(off, _CHUNK)], ibuf)
    for v in range(_CHUNK // 16):
        x = ibuf[pl.ds(v * 16, 16)]
        q = x + x - jnp.where(x >= _MM_HALF, 2 * _MM_HALF - 1, 0)
        ibuf[pl.ds(v * 16, 16)] = q


def _gather_body(table_hbm, idx_hbm, out_hbm, i0, i1, i2, r0, r1, r2, s0, s1, s2):
    wid = lax.axis_index("s") * 2 + lax.axis_index("c")
    base = wid * _W_FULL
    last = jnp.where(wid == 31, _W_LAST, _W_FULL) - _CHUNK

    def off(k):
        return base + jnp.minimum(k * _CHUNK, last)

    ibufs, rbufs, sems = [i0, i1, i2], [r0, r1, r2], [s0, s1, s2]
    hs = [None] * _NCH
    for j in range(_DEPTH - 1):
        _load_q(idx_hbm, ibufs[j], off(j))
        hs[j] = pltpu.async_copy(table_hbm.at[ibufs[j]], rbufs[j], sems[j])
    for k in range(_NCH):
        kk = k + _DEPTH - 1
        if kk < _NCH:
            s = kk % _DEPTH
            _load_q(idx_hbm, ibufs[s], off(kk))
            hs[kk] = pltpu.async_copy(table_hbm.at[ibufs[s]], rbufs[s], sems[s])
        hs[k].wait()
        pltpu.sync_copy(
            rbufs[k % _DEPTH],
            out_hbm.at[pl.ds(off(k), _CHUNK), pl.ds(0, OUT_DIM)],
        )


def _gather_rows(table, idx):
    mesh = plsc.VectorSubcoreMesh(core_axis_name="c", subcore_axis_name="s")
    k = functools.partial(
        pl.kernel,
        mesh=mesh,
        out_type=jax.ShapeDtypeStruct((B, PAD_DIM), jnp.float32),
        compiler_params=pltpu.CompilerParams(use_tc_tiling_on_sc=False),
        scratch_types=[
            pltpu.VMEM((_CHUNK,), jnp.int32),
            pltpu.VMEM((_CHUNK,), jnp.int32),
            pltpu.VMEM((_CHUNK,), jnp.int32),
            pltpu.VMEM((_CHUNK, OUT_DIM), jnp.float32),
            pltpu.VMEM((_CHUNK, OUT_DIM), jnp.float32),
            pltpu.VMEM((_CHUNK, OUT_DIM), jnp.float32),
            pltpu.SemaphoreType.DMA,
            pltpu.SemaphoreType.DMA,
            pltpu.SemaphoreType.DMA,
        ],
    )(_gather_body)
    return k(table, idx)


def kernel(edge_features, neighbors_edge_idxs, W, b):
    table = _transform_table(edge_features, W.T, b.reshape(1, OUT_DIM))
    table = table.reshape(E_ROWS, OUT_DIM)
    idx = neighbors_edge_idxs.astype(jnp.int32)
    return _gather_rows(table, idx)[:, :OUT_DIM]


# speedup vs baseline: 1.2355x; 1.0003x over previous
import functools

import jax
import jax.numpy as jnp
from jax import lax
from jax.experimental import pallas as pl
from jax.experimental.pallas import tpu as pltpu
from jax.experimental.pallas import tpu_sc as plsc

E_ROWS = 150000
IN_DIM = 512
OUT_DIM = 64
PAD_DIM = 128
B = 100000


_MM_ROWS = 3000
_MM_HALF = E_ROWS // 2


def _mm_body(xa_ref, xb_ref, wt_ref, b_ref, o_ref):
    ra = (
        jnp.dot(xa_ref[...], wt_ref[...], preferred_element_type=jnp.float32)
        + b_ref[...]
    )
    rb = (
        jnp.dot(xb_ref[...], wt_ref[...], preferred_element_type=jnp.float32)
        + b_ref[...]
    )
    o_ref[...] = jnp.concatenate([ra, rb], axis=1)


def _transform_table(x, wt, b2d):
    steps = _MM_HALF // _MM_ROWS
    return pl.pallas_call(
        _mm_body,
        grid=(steps,),
        in_specs=[
            pl.BlockSpec((_MM_ROWS, IN_DIM), lambda i: (i, 0)),
            pl.BlockSpec((_MM_ROWS, IN_DIM), lambda i, s=steps: (i + s, 0)),
            pl.BlockSpec((IN_DIM, OUT_DIM), lambda i: (0, 0)),
            pl.BlockSpec((1, OUT_DIM), lambda i: (0, 0)),
        ],
        out_specs=pl.BlockSpec((_MM_ROWS, PAD_DIM), lambda i: (i, 0)),
        out_shape=jax.ShapeDtypeStruct((_MM_HALF, PAD_DIM), jnp.float32),
    )(x, x, wt, b2d)


_CHUNK = 320
_DEPTH = 3
_W_FULL = 3136
_W_LAST = B - 31 * _W_FULL
_NCH = 10


def _load_q(idx_hbm, ibuf, off):
    pltpu.sync_copy(idx_hbm.at[pl.ds(off, _CHUNK)], ibuf)
    for v in range(_CHUNK // 16):
        x = ibuf[pl.ds(v * 16, 16)]
        q = x + x - jnp.where(x >= _MM_HALF, 2 * _MM_HALF - 1, 0)
        ibuf[pl.ds(v * 16, 16)] = q


def _gather_body(table_hbm, idx_hbm, out_hbm, i0, i1, i2, r0, r1, r2, s0, s1, s2):
    wid = lax.axis_index("s") * 2 + lax.axis_index("c")
    base = wid * _W_FULL
    last = jnp.where(wid == 31, _W_LAST, _W_FULL) - _CHUNK

    def off(k):
        return base + jnp.minimum(k * _CHUNK, last)

    ibufs, rbufs, sems = [i0, i1, i2], [r0, r1, r2], [s0, s1, s2]
    hs = [None] * _NCH
    for j in range(_DEPTH - 1):
        _load_q(idx_hbm, ibufs[j], off(j))
        hs[j] = pltpu.async_copy(table_hbm.at[ibufs[j]], rbufs[j], sems[j])
    for k in range(_NCH):
        kk = k + _DEPTH - 1
        if kk < _NCH:
            s = kk % _DEPTH
            _load_q(idx_hbm, ibufs[s], off(kk))
            hs[kk] = pltpu.async_copy(table_hbm.at[ibufs[s]], rbufs[s], sems[s])
        hs[k].wait()
        pltpu.sync_copy(
            rbufs[k % _DEPTH],
            out_hbm.at[pl.ds(off(k), _CHUNK), pl.ds(0, OUT_DIM)],
        )


def _gather_rows(table, idx):
    mesh = plsc.VectorSubcoreMesh(core_axis_name="c", subcore_axis_name="s")
    k = functools.partial(
        pl.kernel,
        mesh=mesh,
        out_type=jax.ShapeDtypeStruct((B, PAD_DIM), jnp.float32),
        compiler_params=pltpu.CompilerParams(use_tc_tiling_on_sc=False),
        scratch_types=[
            pltpu.VMEM((_CHUNK,), jnp.int32),
            pltpu.VMEM((_CHUNK,), jnp.int32),
            pltpu.VMEM((_CHUNK,), jnp.int32),
            pltpu.VMEM((_CHUNK, OUT_DIM), jnp.float32),
            pltpu.VMEM((_CHUNK, OUT_DIM), jnp.float32),
            pltpu.VMEM((_CHUNK, OUT_DIM), jnp.float32),
            pltpu.SemaphoreType.DMA,
            pltpu.SemaphoreType.DMA,
            pltpu.SemaphoreType.DMA,
        ],
    )(_gather_body)
    return k(table, idx)


def kernel(edge_features, neighbors_edge_idxs, W, b):
    table = _transform_table(edge_features, W.T, b.reshape(1, OUT_DIM))
    table = table.reshape(E_ROWS, OUT_DIM)
    idx = neighbors_edge_idxs.astype(jnp.int32)
    return _gather_rows(table, idx)[:, :OUT_DIM]
